# speedup vs baseline: 2.9634x; 2.9634x over previous
import functools

import jax
import jax.numpy as jnp
from jax import lax
from jax.experimental import pallas as pl
from jax.experimental.pallas import tpu as pltpu
from jax.experimental.pallas import tpu_sc as plsc

B_ROWS = 4096
SEQ = 50
D = 128
NUM_WORKERS = 32
B = B_ROWS * SEQ
B_PER_W = B // NUM_WORKERS
CHUNK = 128
NCHUNKS = B_PER_W // CHUNK


def _emb_body(idx_hbm, table_hbm, out_hbm, idx_v, rows_v, sem):
    wid = lax.axis_index("s") * 2 + lax.axis_index("c")
    base = wid * B_PER_W
    pltpu.sync_copy(idx_hbm.at[wid], idx_v)

    def body(j, carry):
        pltpu.async_copy(table_hbm.at[idx_v.at[j]], rows_v, sem).wait()
        pltpu.sync_copy(rows_v, out_hbm.at[pl.ds(base + j * CHUNK, CHUNK)])
        return carry

    lax.fori_loop(0, NCHUNKS, body, 0)


def kernel(input, weight):
    idx = input.reshape(NUM_WORKERS, NCHUNKS, CHUNK).astype(jnp.int32)

    mesh = plsc.VectorSubcoreMesh(core_axis_name="c", subcore_axis_name="s")
    emb = functools.partial(
        pl.kernel,
        mesh=mesh,
        out_type=jax.ShapeDtypeStruct((B, D), jnp.float32),
        scratch_types=[
            pltpu.VMEM((NCHUNKS, CHUNK), jnp.int32),
            pltpu.VMEM((CHUNK, D), jnp.float32),
            pltpu.SemaphoreType.DMA,
        ],
    )(_emb_body)

    out = emb(idx, weight)
    return out.reshape(B_ROWS, SEQ, D)


# speedup vs baseline: 3.3439x; 1.1284x over previous
import functools

import jax
import jax.numpy as jnp
from jax import lax
from jax.experimental import pallas as pl
from jax.experimental.pallas import tpu as pltpu
from jax.experimental.pallas import tpu_sc as plsc

B_ROWS = 4096
SEQ = 50
D = 128
NUM_WORKERS = 32
B = B_ROWS * SEQ
B_PER_W = B // NUM_WORKERS
CHUNK = 128
NCHUNKS = B_PER_W // CHUNK
NBUF = 5


def _emb_body(idx_hbm, table_hbm, out_hbm, idx_v, rows, gsem, osem):
    wid = lax.axis_index("s") * 2 + lax.axis_index("c")
    base = wid * B_PER_W
    pltpu.sync_copy(idx_hbm.at[wid], idx_v)

    for b in range(NBUF):
        pltpu.async_copy(table_hbm.at[idx_v.at[b]], rows.at[b], gsem.at[b])

    def body(i, carry):
        j0 = i * NBUF
        for b in range(NBUF):
            j = j0 + b
            pltpu.make_async_copy(
                table_hbm.at[idx_v.at[j]], rows.at[b], gsem.at[b]).wait()
            pltpu.async_copy(
                rows.at[b], out_hbm.at[pl.ds(base + j * CHUNK, CHUNK)],
                osem.at[b])

            @pl.when(j + NBUF < NCHUNKS)
            def _():
                pltpu.make_async_copy(
                    rows.at[b], out_hbm.at[pl.ds(base + j * CHUNK, CHUNK)],
                    osem.at[b]).wait()
                pltpu.async_copy(
                    table_hbm.at[idx_v.at[j + NBUF]], rows.at[b], gsem.at[b])
        return carry

    lax.fori_loop(0, NCHUNKS // NBUF, body, 0)

    for b in range(NBUF):
        pltpu.make_async_copy(
            rows.at[b], out_hbm.at[pl.ds(base, CHUNK)], osem.at[b]).wait()


def kernel(input, weight):
    idx = input.reshape(NUM_WORKERS, NCHUNKS, CHUNK).astype(jnp.int32)

    mesh = plsc.VectorSubcoreMesh(core_axis_name="c", subcore_axis_name="s")
    emb = functools.partial(
        pl.kernel,
        mesh=mesh,
        out_type=jax.ShapeDtypeStruct((B, D), jnp.float32),
        scratch_types=[
            pltpu.VMEM((NCHUNKS, CHUNK), jnp.int32),
            pltpu.VMEM((NBUF, CHUNK, D), jnp.float32),
            pltpu.SemaphoreType.DMA((NBUF,)),
            pltpu.SemaphoreType.DMA((NBUF,)),
        ],
    )(_emb_body)

    out = emb(idx, weight)
    return out.reshape(B_ROWS, SEQ, D)


# speedup vs baseline: 5.9724x; 1.7860x over previous
import functools

import jax
import jax.numpy as jnp
from jax import lax
from jax.experimental import pallas as pl
from jax.experimental.pallas import tpu as pltpu
from jax.experimental.pallas import tpu_sc as plsc

B_ROWS = 4096
SEQ = 50
D = 128
NUM_WORKERS = 32
S_PER_W = B_ROWS // NUM_WORKERS
NBUF = 8


def _emb_body(idx_hbm, table_hbm, out_hbm, idx_v, rows, gsem, osem):
    wid = lax.axis_index("s") * 2 + lax.axis_index("c")
    base = wid * S_PER_W
    pltpu.sync_copy(idx_hbm.at[wid], idx_v)

    for b in range(NBUF):
        pltpu.async_copy(table_hbm.at[idx_v.at[b]], rows.at[b], gsem.at[b])

    def body(i, carry):
        j0 = i * NBUF
        for b in range(NBUF):
            j = j0 + b
            pltpu.make_async_copy(
                table_hbm.at[idx_v.at[j]], rows.at[b], gsem.at[b]).wait()
            pltpu.async_copy(rows.at[b], out_hbm.at[base + j], osem.at[b])

            @pl.when(j + NBUF < S_PER_W)
            def _():
                pltpu.make_async_copy(
                    rows.at[b], out_hbm.at[base + j], osem.at[b]).wait()
                pltpu.async_copy(
                    table_hbm.at[idx_v.at[j + NBUF]], rows.at[b], gsem.at[b])
        return carry

    lax.fori_loop(0, S_PER_W // NBUF, body, 0)

    for b in range(NBUF):
        pltpu.make_async_copy(
            rows.at[b], out_hbm.at[base], osem.at[b]).wait()


def kernel(input, weight):
    idx = input.reshape(NUM_WORKERS, S_PER_W, SEQ).astype(jnp.int32)

    mesh = plsc.VectorSubcoreMesh(core_axis_name="c", subcore_axis_name="s")
    emb = functools.partial(
        pl.kernel,
        mesh=mesh,
        out_type=jax.ShapeDtypeStruct((B_ROWS, SEQ, D), jnp.float32),
        scratch_types=[
            pltpu.VMEM((S_PER_W, SEQ), jnp.int32),
            pltpu.VMEM((NBUF, SEQ, D), jnp.float32),
            pltpu.SemaphoreType.DMA((NBUF,)),
            pltpu.SemaphoreType.DMA((NBUF,)),
        ],
    )(_emb_body)

    return emb(idx, weight)
